# baseline (speedup 1.0000x reference)
import functools

import jax
import jax.numpy as jnp
from jax import lax
from jax.experimental import pallas as pl
from jax.experimental.pallas import tpu as pltpu
from jax.experimental.pallas import tpu_sc as plsc

N = 10000
D = 128
E = 320000
NC = 2
NS = 16
NW = NC * NS
EPW = E // NW
CH = 80
NCH = 125
NPASS = 5
NCH2 = NCH // NPASS
NBUF = 4
ZCH = 80
ZPT = 624
ZC = 7
ZREM = ZPT - ZC * ZCH
ZTAIL = N - NS * ZPT
NPAD = 10240
RPT = NPAD // NS


def _sc_hist_body(dstr_hbm, ones_hbm, zeros_hbm, out_hbm, dst_v, ones_v, tmp_v, hist_sh):
    c = lax.axis_index("c")
    s = lax.axis_index("s")
    wid = c * NS + s
    pltpu.sync_copy(zeros_hbm, tmp_v)
    pltpu.sync_copy(tmp_v, hist_sh.at[pl.ds(s * RPT, RPT)])
    pltpu.sync_copy(ones_hbm, ones_v)
    plsc.subcore_barrier()

    for p in range(NPASS):
        pltpu.sync_copy(dstr_hbm.at[wid, p], dst_v)

        def body(j, _):
            pltpu.sync_copy(ones_v, hist_sh.at[dst_v.at[j]], add=True)
            return _

        lax.fori_loop(0, NCH2, body, None)

    plsc.subcore_barrier()
    pltpu.sync_copy(hist_sh.at[pl.ds(s * RPT, RPT)], tmp_v)
    pltpu.sync_copy(tmp_v, out_hbm.at[c, pl.ds(s * RPT, RPT)])


def _sc_agg_body(y_hbm, srcr_hbm, dstr_hbm, zeros_hbm, out_hbm,
                 src_v, dst_v, rows_v, agg_sh, sem):
    c = lax.axis_index("c")
    s = lax.axis_index("s")
    wid = c * NS + s
    rows0 = rows_v[0]
    pltpu.sync_copy(zeros_hbm, rows0)

    def zbody(k, _):
        pltpu.sync_copy(rows0.at[pl.ds(0, ZCH)],
                        agg_sh.at[pl.ds(s * ZPT + k * ZCH, ZCH)])
        return _

    lax.fori_loop(0, ZC, zbody, None)
    pltpu.sync_copy(rows0.at[pl.ds(0, ZREM)],
                    agg_sh.at[pl.ds(s * ZPT + ZC * ZCH, ZREM)])

    @pl.when(s == NS - 1)
    def _():
        pltpu.sync_copy(rows0.at[pl.ds(0, ZTAIL)],
                        agg_sh.at[pl.ds(NS * ZPT, ZTAIL)])

    plsc.subcore_barrier()

    for p in range(NPASS):
        pltpu.sync_copy(srcr_hbm.at[wid, p], src_v)
        pltpu.sync_copy(dstr_hbm.at[wid, p], dst_v)
        for j in range(min(NBUF - 1, NCH2)):
            pltpu.async_copy(y_hbm.at[src_v.at[j]], rows_v[j % NBUF],
                             sem[j % NBUF])
        for j in range(NCH2):
            b = j % NBUF
            pltpu.make_async_copy(y_hbm.at[src_v.at[j]], rows_v[b],
                                  sem[b]).wait()
            nxt = j + NBUF - 1
            if nxt < NCH2:
                pltpu.async_copy(y_hbm.at[src_v.at[nxt]], rows_v[nxt % NBUF],
                                 sem[nxt % NBUF])
            pltpu.sync_copy(rows_v[b], agg_sh.at[dst_v.at[j]], add=True)

    plsc.subcore_barrier()

    def wbody(k, _):
        rows = pl.ds(s * ZPT + k * ZCH, ZCH)
        pltpu.sync_copy(agg_sh.at[rows], rows0.at[pl.ds(0, ZCH)])
        pltpu.sync_copy(rows0.at[pl.ds(0, ZCH)], out_hbm.at[c, rows])
        return _

    lax.fori_loop(0, ZC, wbody, None)
    rrows = pl.ds(s * ZPT + ZC * ZCH, ZREM)
    pltpu.sync_copy(agg_sh.at[rrows], rows0.at[pl.ds(0, ZREM)])
    pltpu.sync_copy(rows0.at[pl.ds(0, ZREM)], out_hbm.at[c, rrows])

    @pl.when(s == NS - 1)
    def _():
        rows = pl.ds(NS * ZPT, ZTAIL)
        pltpu.sync_copy(agg_sh.at[rows], rows0.at[pl.ds(0, ZTAIL)])
        pltpu.sync_copy(rows0.at[pl.ds(0, ZTAIL)], out_hbm.at[c, rows])


@functools.cache
def _sc_kernels():
    mesh = plsc.VectorSubcoreMesh(core_axis_name="c", subcore_axis_name="s")
    hist = pl.kernel(
        _sc_hist_body,
        out_type=jax.ShapeDtypeStruct((NC, NPAD), jnp.float32),
        mesh=mesh,
        scratch_types=[
            pltpu.VMEM((NCH2, CH), jnp.int32),
            pltpu.VMEM((CH,), jnp.float32),
            pltpu.VMEM((RPT,), jnp.float32),
            pltpu.VMEM_SHARED((NPAD,), jnp.float32),
        ],
    )
    agg = pl.kernel(
        _sc_agg_body,
        out_type=jax.ShapeDtypeStruct((NC, N, D), jnp.float32),
        mesh=mesh,
        scratch_types=[
            pltpu.VMEM((NCH2, CH), jnp.int32),
            pltpu.VMEM((NCH2, CH), jnp.int32),
            tuple(pltpu.VMEM((CH, D), jnp.float32)
                  for _ in range(NBUF)),
            pltpu.VMEM_SHARED((N, D), jnp.float32),
            tuple(pltpu.SemaphoreType.DMA for _ in range(NBUF)),
        ],
    )
    return hist, agg


_BR = 2048


def _tc1_body(x_ref, w_ref, h_ref, dinv_ref, y_ref):
    deg = h_ref[0][:, None] + h_ref[1][:, None] + 1.0
    dinv = lax.rsqrt(deg)
    dinv_ref[...] = jnp.broadcast_to(dinv, (_BR, 8))
    y_ref[...] = jnp.dot(x_ref[...], w_ref[...],
                         preferred_element_type=jnp.float32) * dinv


def _tc1(x, W1, histp):
    return pl.pallas_call(
        _tc1_body,
        grid=(NPAD // _BR,),
        in_specs=[
            pl.BlockSpec((_BR, D), lambda i: (i, 0)),
            pl.BlockSpec((D, D), lambda i: (0, 0)),
            pl.BlockSpec((NC, _BR), lambda i: (0, i)),
        ],
        out_specs=[
            pl.BlockSpec((_BR, 8), lambda i: (i, 0)),
            pl.BlockSpec((_BR, D), lambda i: (i, 0)),
        ],
        out_shape=[
            jax.ShapeDtypeStruct((NPAD, 8), jnp.float32),
            jax.ShapeDtypeStruct((N, D), jnp.float32),
        ],
    )(x, W1, histp)


def _tc2_body(agg_ref, y1_ref, dinv_ref, b_ref, w_ref, y2_ref):
    dinv = dinv_ref[:, 0:1]
    h = dinv * (agg_ref[0] + agg_ref[1] + y1_ref[...]) + b_ref[...]
    h = jnp.maximum(h, 0.0)
    y2_ref[...] = jnp.dot(h, w_ref[...],
                          preferred_element_type=jnp.float32) * dinv


def _tc2(agg1, y1, dinv8, b1, W2):
    return pl.pallas_call(
        _tc2_body,
        grid=(NPAD // _BR,),
        in_specs=[
            pl.BlockSpec((NC, _BR, D), lambda i: (0, i, 0)),
            pl.BlockSpec((_BR, D), lambda i: (i, 0)),
            pl.BlockSpec((_BR, 8), lambda i: (i, 0)),
            pl.BlockSpec((1, D), lambda i: (0, 0)),
            pl.BlockSpec((D, D), lambda i: (0, 0)),
        ],
        out_specs=pl.BlockSpec((_BR, D), lambda i: (i, 0)),
        out_shape=jax.ShapeDtypeStruct((N, D), jnp.float32),
    )(agg1, y1, dinv8, b1, W2)


def _tc3_body(agg_ref, y2_ref, dinv_ref, b_ref, o_ref):
    dinv = dinv_ref[:, 0:1]
    o_ref[...] = dinv * (agg_ref[0] + agg_ref[1] + y2_ref[...]) + b_ref[...]


def _tc3(agg2, y2, dinv8, b2):
    return pl.pallas_call(
        _tc3_body,
        grid=(NPAD // _BR,),
        in_specs=[
            pl.BlockSpec((NC, _BR, D), lambda i: (0, i, 0)),
            pl.BlockSpec((_BR, D), lambda i: (i, 0)),
            pl.BlockSpec((_BR, 8), lambda i: (i, 0)),
            pl.BlockSpec((1, D), lambda i: (0, 0)),
        ],
        out_specs=pl.BlockSpec((_BR, D), lambda i: (i, 0)),
        out_shape=jax.ShapeDtypeStruct((N, D), jnp.float32),
    )(agg2, y2, dinv8, b2)


def kernel(x, edge_index, W1, b1, W2, b2):
    src = edge_index[0].astype(jnp.int32)
    dst = edge_index[1].astype(jnp.int32)
    srcr = src.reshape(NW, NPASS, NCH2, CH)
    dstr = dst.reshape(NW, NPASS, NCH2, CH)
    ones1 = jnp.ones((CH,), jnp.float32)
    zeros1 = jnp.zeros((RPT,), jnp.float32)
    zerosd = jnp.zeros((CH, D), jnp.float32)

    sc_hist, sc_agg = _sc_kernels()
    histp = sc_hist(dstr, ones1, zeros1)
    dinv8, y1 = _tc1(x, W1, histp)
    agg1 = sc_agg(y1, srcr, dstr, zerosd)
    y2 = _tc2(agg1, y1, dinv8, b1.reshape(1, D), W2)
    agg2 = sc_agg(y2, srcr, dstr, zerosd)
    return _tc3(agg2, y2, dinv8, b2.reshape(1, D))


# speedup vs baseline: 1.0094x; 1.0094x over previous
import functools

import jax
import jax.numpy as jnp
from jax import lax
from jax.experimental import pallas as pl
from jax.experimental.pallas import tpu as pltpu
from jax.experimental.pallas import tpu_sc as plsc

N = 10000
D = 128
E = 320000
NC = 2
NS = 16
NW = NC * NS
EPW = E // NW
CH = 100
NCH = 100
NPASS = 4
NCH2 = NCH // NPASS
NBUF = 3
ZCH = 96
ZPT = 624
ZC = 6
ZREM = ZPT - ZC * ZCH
ZTAIL = N - NS * ZPT
NPAD = 10240
RPT = NPAD // NS


def _sc_hist_body(dstr_hbm, ones_hbm, zeros_hbm, out_hbm, dst_v, ones_v, tmp_v, hist_sh):
    c = lax.axis_index("c")
    s = lax.axis_index("s")
    wid = c * NS + s
    pltpu.sync_copy(zeros_hbm, tmp_v)
    pltpu.sync_copy(tmp_v, hist_sh.at[pl.ds(s * RPT, RPT)])
    pltpu.sync_copy(ones_hbm, ones_v)
    plsc.subcore_barrier()

    for p in range(NPASS):
        pltpu.sync_copy(dstr_hbm.at[wid, p], dst_v)

        def body(j, _):
            pltpu.sync_copy(ones_v, hist_sh.at[dst_v.at[j]], add=True)
            return _

        lax.fori_loop(0, NCH2, body, None)

    plsc.subcore_barrier()
    pltpu.sync_copy(hist_sh.at[pl.ds(s * RPT, RPT)], tmp_v)
    pltpu.sync_copy(tmp_v, out_hbm.at[c, pl.ds(s * RPT, RPT)])


def _sc_agg_body(y_hbm, srcr_hbm, dstr_hbm, zeros_hbm, out_hbm,
                 src_v, dst_v, rows_v, agg_sh, sem):
    c = lax.axis_index("c")
    s = lax.axis_index("s")
    wid = c * NS + s
    rows0 = rows_v[0]
    pltpu.sync_copy(zeros_hbm, rows0)

    def zbody(k, _):
        pltpu.sync_copy(rows0.at[pl.ds(0, ZCH)],
                        agg_sh.at[pl.ds(s * ZPT + k * ZCH, ZCH)])
        return _

    lax.fori_loop(0, ZC, zbody, None)
    pltpu.sync_copy(rows0.at[pl.ds(0, ZREM)],
                    agg_sh.at[pl.ds(s * ZPT + ZC * ZCH, ZREM)])

    @pl.when(s == NS - 1)
    def _():
        pltpu.sync_copy(rows0.at[pl.ds(0, ZTAIL)],
                        agg_sh.at[pl.ds(NS * ZPT, ZTAIL)])

    plsc.subcore_barrier()

    for p in range(NPASS):
        pltpu.sync_copy(srcr_hbm.at[wid, p], src_v)
        pltpu.sync_copy(dstr_hbm.at[wid, p], dst_v)
        for j in range(min(NBUF - 1, NCH2)):
            pltpu.async_copy(y_hbm.at[src_v.at[j]], rows_v[j % NBUF],
                             sem[j % NBUF])
        for j in range(NCH2):
            b = j % NBUF
            pltpu.make_async_copy(y_hbm.at[src_v.at[j]], rows_v[b],
                                  sem[b]).wait()
            nxt = j + NBUF - 1
            if nxt < NCH2:
                pltpu.async_copy(y_hbm.at[src_v.at[nxt]], rows_v[nxt % NBUF],
                                 sem[nxt % NBUF])
            pltpu.sync_copy(rows_v[b], agg_sh.at[dst_v.at[j]], add=True)

    plsc.subcore_barrier()

    def wbody(k, _):
        rows = pl.ds(s * ZPT + k * ZCH, ZCH)
        pltpu.sync_copy(agg_sh.at[rows], rows0.at[pl.ds(0, ZCH)])
        pltpu.sync_copy(rows0.at[pl.ds(0, ZCH)], out_hbm.at[c, rows])
        return _

    lax.fori_loop(0, ZC, wbody, None)
    rrows = pl.ds(s * ZPT + ZC * ZCH, ZREM)
    pltpu.sync_copy(agg_sh.at[rrows], rows0.at[pl.ds(0, ZREM)])
    pltpu.sync_copy(rows0.at[pl.ds(0, ZREM)], out_hbm.at[c, rrows])

    @pl.when(s == NS - 1)
    def _():
        rows = pl.ds(NS * ZPT, ZTAIL)
        pltpu.sync_copy(agg_sh.at[rows], rows0.at[pl.ds(0, ZTAIL)])
        pltpu.sync_copy(rows0.at[pl.ds(0, ZTAIL)], out_hbm.at[c, rows])


@functools.cache
def _sc_kernels():
    mesh = plsc.VectorSubcoreMesh(core_axis_name="c", subcore_axis_name="s")
    hist = pl.kernel(
        _sc_hist_body,
        out_type=jax.ShapeDtypeStruct((NC, NPAD), jnp.float32),
        mesh=mesh,
        scratch_types=[
            pltpu.VMEM((NCH2, CH), jnp.int32),
            pltpu.VMEM((CH,), jnp.float32),
            pltpu.VMEM((RPT,), jnp.float32),
            pltpu.VMEM_SHARED((NPAD,), jnp.float32),
        ],
    )
    agg = pl.kernel(
        _sc_agg_body,
        out_type=jax.ShapeDtypeStruct((NC, N, D), jnp.float32),
        mesh=mesh,
        scratch_types=[
            pltpu.VMEM((NCH2, CH), jnp.int32),
            pltpu.VMEM((NCH2, CH), jnp.int32),
            tuple(pltpu.VMEM((CH, D), jnp.float32)
                  for _ in range(NBUF)),
            pltpu.VMEM_SHARED((N, D), jnp.float32),
            tuple(pltpu.SemaphoreType.DMA for _ in range(NBUF)),
        ],
    )
    return hist, agg


_BR = 2048


def _tc1_body(x_ref, w_ref, h_ref, dinv_ref, y_ref):
    deg = h_ref[0][:, None] + h_ref[1][:, None] + 1.0
    dinv = lax.rsqrt(deg)
    dinv_ref[...] = jnp.broadcast_to(dinv, (_BR, 8))
    y_ref[...] = jnp.dot(x_ref[...], w_ref[...],
                         preferred_element_type=jnp.float32) * dinv


def _tc1(x, W1, histp):
    return pl.pallas_call(
        _tc1_body,
        grid=(NPAD // _BR,),
        in_specs=[
            pl.BlockSpec((_BR, D), lambda i: (i, 0)),
            pl.BlockSpec((D, D), lambda i: (0, 0)),
            pl.BlockSpec((NC, _BR), lambda i: (0, i)),
        ],
        out_specs=[
            pl.BlockSpec((_BR, 8), lambda i: (i, 0)),
            pl.BlockSpec((_BR, D), lambda i: (i, 0)),
        ],
        out_shape=[
            jax.ShapeDtypeStruct((NPAD, 8), jnp.float32),
            jax.ShapeDtypeStruct((N, D), jnp.float32),
        ],
    )(x, W1, histp)


def _tc2_body(agg_ref, y1_ref, dinv_ref, b_ref, w_ref, y2_ref):
    dinv = dinv_ref[:, 0:1]
    h = dinv * (agg_ref[0] + agg_ref[1] + y1_ref[...]) + b_ref[...]
    h = jnp.maximum(h, 0.0)
    y2_ref[...] = jnp.dot(h, w_ref[...],
                          preferred_element_type=jnp.float32) * dinv


def _tc2(agg1, y1, dinv8, b1, W2):
    return pl.pallas_call(
        _tc2_body,
        grid=(NPAD // _BR,),
        in_specs=[
            pl.BlockSpec((NC, _BR, D), lambda i: (0, i, 0)),
            pl.BlockSpec((_BR, D), lambda i: (i, 0)),
            pl.BlockSpec((_BR, 8), lambda i: (i, 0)),
            pl.BlockSpec((1, D), lambda i: (0, 0)),
            pl.BlockSpec((D, D), lambda i: (0, 0)),
        ],
        out_specs=pl.BlockSpec((_BR, D), lambda i: (i, 0)),
        out_shape=jax.ShapeDtypeStruct((N, D), jnp.float32),
    )(agg1, y1, dinv8, b1, W2)


def _tc3_body(agg_ref, y2_ref, dinv_ref, b_ref, o_ref):
    dinv = dinv_ref[:, 0:1]
    o_ref[...] = dinv * (agg_ref[0] + agg_ref[1] + y2_ref[...]) + b_ref[...]


def _tc3(agg2, y2, dinv8, b2):
    return pl.pallas_call(
        _tc3_body,
        grid=(NPAD // _BR,),
        in_specs=[
            pl.BlockSpec((NC, _BR, D), lambda i: (0, i, 0)),
            pl.BlockSpec((_BR, D), lambda i: (i, 0)),
            pl.BlockSpec((_BR, 8), lambda i: (i, 0)),
            pl.BlockSpec((1, D), lambda i: (0, 0)),
        ],
        out_specs=pl.BlockSpec((_BR, D), lambda i: (i, 0)),
        out_shape=jax.ShapeDtypeStruct((N, D), jnp.float32),
    )(agg2, y2, dinv8, b2)


def kernel(x, edge_index, W1, b1, W2, b2):
    src = edge_index[0].astype(jnp.int32)
    dst = edge_index[1].astype(jnp.int32)
    srcr = src.reshape(NW, NPASS, NCH2, CH)
    dstr = dst.reshape(NW, NPASS, NCH2, CH)
    ones1 = jnp.ones((CH,), jnp.float32)
    zeros1 = jnp.zeros((RPT,), jnp.float32)
    zerosd = jnp.zeros((CH, D), jnp.float32)

    sc_hist, sc_agg = _sc_kernels()
    histp = sc_hist(dstr, ones1, zeros1)
    dinv8, y1 = _tc1(x, W1, histp)
    agg1 = sc_agg(y1, srcr, dstr, zerosd)
    y2 = _tc2(agg1, y1, dinv8, b1.reshape(1, D), W2)
    agg2 = sc_agg(y2, srcr, dstr, zerosd)
    return _tc3(agg2, y2, dinv8, b2.reshape(1, D))
